# baseline (speedup 1.0000x reference)
import functools

import jax
import jax.numpy as jnp
from jax import lax
from jax.experimental import pallas as pl
from jax.experimental.pallas import tpu as pltpu
from jax.experimental.pallas import tpu_sc as plsc

_N = 4096
_SCALE = 0.125
_L = 16
_NROWBLK = 64
_NCHUNK = _NROWBLK * 128 // _L
_LO = 241
_HI = 271


def _bucket_values(d, wtab):
    rel_pos = d - (_N - 1)
    n = -rel_pos
    ret = (n < 0).astype(jnp.int32) * 16
    n = jnp.abs(n)
    is_small = n < 8
    safe_n = jnp.maximum(n, 1)
    e = (lax.bitcast_convert_type(safe_n.astype(jnp.float32), jnp.int32) >> 23) - 127
    val_if_large = 2 * e + 2 + (safe_n * safe_n >= (1 << (2 * e + 1))).astype(jnp.int32)
    val_if_large = jnp.minimum(val_if_large, 15)
    bucket = ret + jnp.where(is_small, n, val_if_large)
    zero = jnp.zeros((_L,), jnp.int32)
    return plsc.load_gather(wtab, [bucket, zero]) * _SCALE


_MESH = plsc.VectorSubcoreMesh(core_axis_name="c", subcore_axis_name="s")


@functools.partial(
    pl.kernel,
    mesh=_MESH,
    out_type=jax.ShapeDtypeStruct((_N // 8, _N // 128, 8, 128), jnp.float32),
    scratch_types=[
        pltpu.VMEM((8, _NROWBLK, 128), jnp.float32),
        pltpu.VMEM((32, 1), jnp.float32),
        pltpu.SemaphoreType.DMA,
    ],
    compiler_params=pltpu.CompilerParams(
        use_tc_tiling_on_sc=False,
        needs_layout_passes=False,
        disable_bounds_checks=True,
        disable_semaphore_checks=True,
        skip_device_barrier=True,
    ),
)
def _sc_bias(w_hbm, out_hbm, vshift, wtab, sem):
    cid = lax.axis_index("c")
    sid = lax.axis_index("s")
    pltpu.sync_copy(w_hbm, wtab)

    zeros = jnp.zeros((_L,), jnp.int32)
    c15 = plsc.load_gather(wtab, [jnp.full((_L,), 15, jnp.int32), zeros]) * _SCALE
    c31 = plsc.load_gather(wtab, [jnp.full((_L,), 31, jnp.int32), zeros]) * _SCALE
    lane = lax.iota(jnp.int32, _L)

    def fill(u, k, val):
        vshift[u, k // 8, pl.ds(_L * (k % 8), _L)] = val

    def row_copy(u, k):
        m = k + 16 * cid
        i = (_N - 1) - (sid + 16 * u) - 128 * m
        return pltpu.make_async_copy(
            vshift.at[u, pl.ds(m, 32), :],
            out_hbm.at[i // 8, :, i % 8],
            sem,
        )

    def fill_u(u, carry):
        def fill_lo(k, c):
            fill(u, k, c15)
            return c

        def fill_mid(k, c):
            d = _L * k + lane + sid + 16 * u
            fill(u, k, _bucket_values(d, wtab))
            return c

        def fill_hi(k, c):
            fill(u, k, c31)
            return c

        lax.fori_loop(0, _LO, fill_lo, 0)
        lax.fori_loop(_LO, _HI, fill_mid, 0)
        lax.fori_loop(_HI, _NCHUNK, fill_hi, 0)

        def fire_row(k, c):
            row_copy(u, k).start()
            return c

        lax.fori_loop(0, 16, fire_row, 0)
        return carry

    lax.fori_loop(0, 8, fill_u, 0)

    def drain_u(u, carry):
        def drain_row(k, c):
            row_copy(u, k).wait()
            return c

        lax.fori_loop(0, 16, drain_row, 0)
        return carry

    lax.fori_loop(0, 8, drain_u, 0)


def kernel(x, W):
    del x
    o4 = _sc_bias(W)
    return o4.transpose(0, 2, 1, 3).reshape(_N, _N)


# speedup vs baseline: 1.0195x; 1.0195x over previous
import functools

import jax
import jax.numpy as jnp
from jax import lax
from jax.experimental import pallas as pl
from jax.experimental.pallas import tpu as pltpu
from jax.experimental.pallas import tpu_sc as plsc

_N = 4096
_SCALE = 0.125
_L = 16
_NROWBLK = 64
_NCHUNK = _NROWBLK * 128 // _L
_LO = 241
_HI = 271


def _bucket_values(d, wtab):
    rel_pos = d - (_N - 1)
    n = -rel_pos
    ret = (n < 0).astype(jnp.int32) * 16
    n = jnp.abs(n)
    is_small = n < 8
    safe_n = jnp.maximum(n, 1)
    e = (lax.bitcast_convert_type(safe_n.astype(jnp.float32), jnp.int32) >> 23) - 127
    val_if_large = 2 * e + 2 + (safe_n * safe_n >= (1 << (2 * e + 1))).astype(jnp.int32)
    val_if_large = jnp.minimum(val_if_large, 15)
    bucket = ret + jnp.where(is_small, n, val_if_large)
    return plsc.load_gather(wtab, [bucket]) * _SCALE


_MESH = plsc.VectorSubcoreMesh(core_axis_name="c", subcore_axis_name="s")


@functools.partial(
    pl.kernel,
    mesh=_MESH,
    out_type=jax.ShapeDtypeStruct((_N // 8, _N // 128, 8, 128), jnp.float32),
    scratch_types=[
        pltpu.VMEM((8, _NROWBLK, 128), jnp.float32),
        pltpu.VMEM((32,), jnp.float32),
        pltpu.SemaphoreType.DMA,
    ],
    compiler_params=pltpu.CompilerParams(
        use_tc_tiling_on_sc=False,
        needs_layout_passes=False,
    ),
)
def _sc_bias(w_hbm, out_hbm, vshift, wtab, sem):
    cid = lax.axis_index("c")
    sid = lax.axis_index("s")
    pltpu.sync_copy(w_hbm, wtab)

    c15 = plsc.load_gather(wtab, [jnp.full((_L,), 15, jnp.int32)]) * _SCALE
    c31 = plsc.load_gather(wtab, [jnp.full((_L,), 31, jnp.int32)]) * _SCALE
    lane = lax.iota(jnp.int32, _L)

    def fill(u, k, val):
        vshift[u, k // 8, pl.ds(_L * (k % 8), _L)] = val

    def row_copy(u, k):
        m = k + 16 * cid
        i = (_N - 1) - (sid + 16 * u) - 128 * m
        return pltpu.make_async_copy(
            vshift.at[u, pl.ds(m, 32), :],
            out_hbm.at[i // 8, :, i % 8],
            sem,
        )

    def fill_u(u, carry):
        def fill_lo(k, c):
            fill(u, k, c15)
            return c

        def fill_mid(k, c):
            d = _L * k + lane + sid + 16 * u
            fill(u, k, _bucket_values(d, wtab))
            return c

        def fill_hi(k, c):
            fill(u, k, c31)
            return c

        lax.fori_loop(0, _LO, fill_lo, 0)
        lax.fori_loop(_LO, _HI, fill_mid, 0)
        lax.fori_loop(_HI, _NCHUNK, fill_hi, 0)

        def fire_row(k, c):
            row_copy(u, k).start()
            return c

        lax.fori_loop(0, 16, fire_row, 0)
        return carry

    lax.fori_loop(0, 8, fill_u, 0)

    def drain_u(u, carry):
        def drain_row(k, c):
            row_copy(u, k).wait()
            return c

        lax.fori_loop(0, 16, drain_row, 0)
        return carry

    lax.fori_loop(0, 8, drain_u, 0)


def kernel(x, W):
    del x
    o4 = _sc_bias(W.reshape(32))
    return o4.transpose(0, 2, 1, 3).reshape(_N, _N)
